# Initial kernel scaffold; baseline (speedup 1.0000x reference)
import jax
import jax.numpy as jnp
from jax.experimental import pallas as pl


def kernel(x, selected_experts, routing_weights, Wg, Wu, Wd):
    raise NotImplementedError("write your pallas kernel here")


# speedup vs baseline: 6.3066x; 6.3066x over previous
import functools

import jax
import jax.numpy as jnp
from jax import lax
from jax.experimental import pallas as pl
from jax.experimental.pallas import tpu as pltpu
from jax.experimental.pallas import tpu_sc as plsc

H = 1024
I = 1024
E = 64
T = 2048
IB = 256
NI = I // IB
CHUNK = 128
NC, NS = 2, 16
NW = NC * NS
P = 2816
BPW = P // NW
TRASH = T


def _gelu(v):
    return 0.5 * v * (1.0 + jnp.tanh(jnp.sqrt(2.0 / jnp.pi) * (v + 0.044715 * v ** 3)))


_SC_CACHE = {}


def _sc_kernels():
    if "gather" in _SC_CACHE:
        return _SC_CACHE["gather"], _SC_CACHE["scatter"]
    mesh = plsc.VectorSubcoreMesh(core_axis_name="c", subcore_axis_name="s")
    scratch = [
        pltpu.VMEM((BPW,), jnp.int32),
        pltpu.VMEM((BPW, H), jnp.float32),
        pltpu.SemaphoreType.DMA,
    ]

    @functools.partial(
        pl.kernel, mesh=mesh,
        out_type=jax.ShapeDtypeStruct((P, H), jnp.float32),
        scratch_types=scratch,
    )
    def _sc_gather(table_hbm, idx_hbm, out_hbm, idx_v, rows_v, sem):
        wid = lax.axis_index("s") * NC + lax.axis_index("c")
        base = wid * BPW
        pltpu.sync_copy(idx_hbm.at[pl.ds(base, BPW)], idx_v)
        pltpu.async_copy(table_hbm.at[idx_v], rows_v, sem).wait()
        pltpu.sync_copy(rows_v, out_hbm.at[pl.ds(base, BPW)])

    @functools.partial(
        pl.kernel, mesh=mesh,
        out_type=jax.ShapeDtypeStruct((T + P, H), jnp.float32),
        scratch_types=scratch,
    )
    def _sc_scatter(rows_hbm, idx_hbm, out_hbm, idx_v, rows_v, sem):
        wid = lax.axis_index("s") * NC + lax.axis_index("c")
        base = wid * BPW
        pltpu.sync_copy(idx_hbm.at[pl.ds(base, BPW)], idx_v)
        pltpu.sync_copy(rows_hbm.at[pl.ds(base, BPW)], rows_v)
        pltpu.async_copy(rows_v, out_hbm.at[idx_v], sem).wait()

    _SC_CACHE["gather"] = _sc_gather
    _SC_CACHE["scatter"] = _sc_scatter
    return _sc_gather, _sc_scatter


def _tc_body(poff_ref, xs_ref, ws_ref, wg_ref, wu_ref, wd_ref, ys_ref):
    e = pl.program_id(0)
    it = pl.program_id(1)
    start = poff_ref[e]
    size = poff_ref[e + 1] - start
    nch = (size + CHUNK - 1) // CHUNK
    wg = wg_ref[0]
    wu = wu_ref[0]
    wd = wd_ref[0]

    def chunk(i, carry):
        base = pl.multiple_of(start + i * CHUNK, 8)
        rows = xs_ref[pl.ds(base, CHUNK), :]
        g = lax.dot_general(rows, wg, (((1,), (1,)), ((), ())),
                            preferred_element_type=jnp.float32)
        u = lax.dot_general(rows, wu, (((1,), (1,)), ((), ())),
                            preferred_element_type=jnp.float32)
        h = _gelu(g) * u
        part = lax.dot_general(h, wd, (((1,), (1,)), ((), ())),
                               preferred_element_type=jnp.float32)
        part = part * ws_ref[pl.ds(base, CHUNK), :]

        @pl.when(it == 0)
        def _():
            ys_ref[pl.ds(base, CHUNK), :] = part

        @pl.when(it > 0)
        def _():
            ys_ref[pl.ds(base, CHUNK), :] += part

        return carry

    lax.fori_loop(0, nch, chunk, 0)


def _tc_grouped(poff, xs, ws, Wg, Wu, Wd):
    return pl.pallas_call(
        _tc_body,
        grid=(E, NI),
        in_specs=[
            pl.BlockSpec(memory_space=pltpu.SMEM),
            pl.BlockSpec((P, H), lambda e, it: (0, 0)),
            pl.BlockSpec((P, 1), lambda e, it: (0, 0)),
            pl.BlockSpec((1, IB, H), lambda e, it: (e, it, 0)),
            pl.BlockSpec((1, IB, H), lambda e, it: (e, it, 0)),
            pl.BlockSpec((1, H, IB), lambda e, it: (e, 0, it)),
        ],
        out_specs=pl.BlockSpec((P, H), lambda e, it: (0, 0)),
        out_shape=jax.ShapeDtypeStruct((P, H), jnp.float32),
        compiler_params=pltpu.CompilerParams(
            dimension_semantics=("arbitrary", "arbitrary")),
    )(poff, xs, ws, Wg, Wu, Wd)


def kernel(x, selected_experts, routing_weights, Wg, Wu, Wd):
    fe = selected_experts.reshape(-1).astype(jnp.int32)
    fw = routing_weights.reshape(-1).astype(jnp.float32)

    oh = (fe[:, None] == jnp.arange(E, dtype=jnp.int32)[None, :]).astype(jnp.int32)
    csum = jnp.cumsum(oh, axis=0)
    counts = csum[-1]
    rank = jnp.sum(oh * csum, axis=1) - 1
    pcounts = ((counts + 7) // 8) * 8
    poff = jnp.concatenate([jnp.zeros((1,), jnp.int32),
                            jnp.cumsum(pcounts).astype(jnp.int32)])
    pos = jnp.take(poff, fe) + rank

    tok = jnp.arange(T, dtype=jnp.int32)
    src = jnp.full((P,), TRASH, jnp.int32).at[pos].set(tok)
    dest = jnp.where(src == TRASH, TRASH + jnp.arange(P, dtype=jnp.int32), src)
    ws = jnp.zeros((P,), jnp.float32).at[pos].set(fw).reshape(P, 1)

    x_ext = jnp.concatenate([x, jnp.zeros((8, H), jnp.float32)], axis=0)
    sc_gather, sc_scatter = _sc_kernels()
    xs = sc_gather(x_ext, src)
    ys = _tc_grouped(poff, xs, ws, Wg, Wu, Wd)
    out_ext = sc_scatter(ys, dest)
    return out_ext[:T]


# speedup vs baseline: 8.7060x; 1.3805x over previous
import functools

import jax
import jax.numpy as jnp
from jax import lax
from jax.experimental import pallas as pl
from jax.experimental.pallas import tpu as pltpu
from jax.experimental.pallas import tpu_sc as plsc

H = 1024
I = 1024
E = 64
T = 2048
IB = 256
NI = I // IB
CHUNK = 64
NC, NS = 2, 16
NW = NC * NS
P = 2560
BPW = P // NW
TRASH = T


def _gelu(v):
    return 0.5 * v * (1.0 + jnp.tanh(jnp.sqrt(2.0 / jnp.pi) * (v + 0.044715 * v ** 3)))


_SC_CACHE = {}


def _sc_kernels():
    if "gather" in _SC_CACHE:
        return _SC_CACHE["gather"], _SC_CACHE["scatter"]
    mesh = plsc.VectorSubcoreMesh(core_axis_name="c", subcore_axis_name="s")
    scratch = [
        pltpu.VMEM((BPW,), jnp.int32),
        pltpu.VMEM((BPW, H), jnp.float32),
        pltpu.SemaphoreType.DMA,
    ]

    @functools.partial(
        pl.kernel, mesh=mesh,
        out_type=jax.ShapeDtypeStruct((P, H), jnp.float32),
        scratch_types=scratch,
    )
    def _sc_gather(table_hbm, idx_hbm, out_hbm, idx_v, rows_v, sem):
        wid = lax.axis_index("s") * NC + lax.axis_index("c")
        base = wid * BPW
        pltpu.sync_copy(idx_hbm.at[pl.ds(base, BPW)], idx_v)
        pltpu.async_copy(table_hbm.at[idx_v], rows_v, sem).wait()
        pltpu.sync_copy(rows_v, out_hbm.at[pl.ds(base, BPW)])

    @functools.partial(
        pl.kernel, mesh=mesh,
        out_type=jax.ShapeDtypeStruct((T + P, H), jnp.float32),
        scratch_types=scratch,
    )
    def _sc_scatter(rows_hbm, idx_hbm, out_hbm, idx_v, rows_v, sem):
        wid = lax.axis_index("s") * NC + lax.axis_index("c")
        base = wid * BPW
        pltpu.sync_copy(idx_hbm.at[pl.ds(base, BPW)], idx_v)
        pltpu.sync_copy(rows_hbm.at[pl.ds(base, BPW)], rows_v)
        pltpu.async_copy(rows_v, out_hbm.at[idx_v], sem).wait()

    _SC_CACHE["gather"] = _sc_gather
    _SC_CACHE["scatter"] = _sc_scatter
    return _sc_gather, _sc_scatter


def _tc_body(poff_ref, xs_ref, ws_ref, wg_ref, wu_ref, wd_ref, ys_ref):
    e = pl.program_id(0)
    start = poff_ref[e]
    size = poff_ref[e + 1] - start
    nch = (size + CHUNK - 1) // CHUNK
    wg = wg_ref[0]
    wu = wu_ref[0]
    wd = wd_ref[0]

    def chunk(i, carry):
        base = pl.multiple_of(start + i * CHUNK, 8)
        rows = xs_ref[pl.ds(base, CHUNK), :]
        g = lax.dot_general(rows, wg, (((1,), (1,)), ((), ())),
                            preferred_element_type=jnp.float32)
        u = lax.dot_general(rows, wu, (((1,), (1,)), ((), ())),
                            preferred_element_type=jnp.float32)
        h = _gelu(g) * u
        part = lax.dot_general(h, wd, (((1,), (1,)), ((), ())),
                               preferred_element_type=jnp.float32)
        ys_ref[pl.ds(base, CHUNK), :] = part * ws_ref[pl.ds(base, CHUNK), :]
        return carry

    lax.fori_loop(0, nch, chunk, 0)


def _tc_grouped(poff, xs, ws, Wg, Wu, Wd):
    return pl.pallas_call(
        _tc_body,
        grid=(E,),
        in_specs=[
            pl.BlockSpec(memory_space=pltpu.SMEM),
            pl.BlockSpec((P, H), lambda e: (0, 0)),
            pl.BlockSpec((P, 1), lambda e: (0, 0)),
            pl.BlockSpec((1, I, H), lambda e: (e, 0, 0)),
            pl.BlockSpec((1, I, H), lambda e: (e, 0, 0)),
            pl.BlockSpec((1, H, I), lambda e: (e, 0, 0)),
        ],
        out_specs=pl.BlockSpec((P, H), lambda e: (0, 0)),
        out_shape=jax.ShapeDtypeStruct((P, H), jnp.float32),
        compiler_params=pltpu.CompilerParams(
            dimension_semantics=("arbitrary",)),
    )(poff, xs, ws, Wg, Wu, Wd)


def kernel(x, selected_experts, routing_weights, Wg, Wu, Wd):
    fe = selected_experts.reshape(-1).astype(jnp.int32)
    fw = routing_weights.reshape(-1).astype(jnp.float32)

    oh = (fe[:, None] == jnp.arange(E, dtype=jnp.int32)[None, :]).astype(jnp.int32)
    csum = jnp.cumsum(oh, axis=0)
    counts = csum[-1]
    rank = jnp.sum(oh * csum, axis=1) - 1
    pcounts = ((counts + 7) // 8) * 8
    poff = jnp.concatenate([jnp.zeros((1,), jnp.int32),
                            jnp.cumsum(pcounts).astype(jnp.int32)])
    pos = jnp.take(poff, fe) + rank

    tok = jnp.arange(T, dtype=jnp.int32)
    src = jnp.full((P,), TRASH, jnp.int32).at[pos].set(tok)
    dest = jnp.where(src == TRASH, TRASH + jnp.arange(P, dtype=jnp.int32), src)
    ws = jnp.zeros((P,), jnp.float32).at[pos].set(fw).reshape(P, 1)

    x_ext = jnp.concatenate([x, jnp.zeros((8, H), jnp.float32)], axis=0)
    sc_gather, sc_scatter = _sc_kernels()
    xs = sc_gather(x_ext, src)
    ys = _tc_grouped(poff, xs, ws, Wg, Wu, Wd)
    out_ext = sc_scatter(ys, dest)
    return out_ext[:T]


# speedup vs baseline: 8.9293x; 1.0257x over previous
import functools

import jax
import jax.numpy as jnp
from jax import lax
from jax.experimental import pallas as pl
from jax.experimental.pallas import tpu as pltpu
from jax.experimental.pallas import tpu_sc as plsc

H = 1024
I = 1024
E = 64
T = 2048
IB = 256
NI = I // IB
CHUNK = 64
NC, NS = 2, 16
NW = NC * NS
P = 2560
BPW = P // NW
TRASH = T


def _gelu(v):
    return 0.5 * v * (1.0 + jnp.tanh(jnp.sqrt(2.0 / jnp.pi) * (v + 0.044715 * v ** 3)))


_SC_CACHE = {}


def _sc_kernels():
    if "gather" in _SC_CACHE:
        return _SC_CACHE["gather"], _SC_CACHE["scatter"]
    mesh = plsc.VectorSubcoreMesh(core_axis_name="c", subcore_axis_name="s")
    scratch = [
        pltpu.VMEM((BPW,), jnp.int32),
        pltpu.VMEM((BPW, H), jnp.float32),
        pltpu.SemaphoreType.DMA,
    ]

    @functools.partial(
        pl.kernel, mesh=mesh,
        out_type=jax.ShapeDtypeStruct((P, H), jnp.float32),
        scratch_types=scratch,
    )
    def _sc_gather(table_hbm, idx_hbm, out_hbm, idx_v, rows_v, sem):
        wid = lax.axis_index("s") * NC + lax.axis_index("c")
        base = wid * BPW
        pltpu.sync_copy(idx_hbm.at[pl.ds(base, BPW)], idx_v)
        pltpu.async_copy(table_hbm.at[idx_v], rows_v, sem).wait()
        pltpu.sync_copy(rows_v, out_hbm.at[pl.ds(base, BPW)])

    @functools.partial(
        pl.kernel, mesh=mesh,
        out_type=jax.ShapeDtypeStruct((T + P, H), jnp.float32),
        scratch_types=scratch,
    )
    def _sc_scatter(rows_hbm, idx_hbm, out_hbm, idx_v, rows_v, sem):
        wid = lax.axis_index("s") * NC + lax.axis_index("c")
        base = wid * BPW
        pltpu.sync_copy(idx_hbm.at[pl.ds(base, BPW)], idx_v)
        pltpu.sync_copy(rows_hbm.at[pl.ds(base, BPW)], rows_v)
        pltpu.async_copy(rows_v, out_hbm.at[idx_v], sem).wait()

    _SC_CACHE["gather"] = _sc_gather
    _SC_CACHE["scatter"] = _sc_scatter
    return _sc_gather, _sc_scatter


def _tc_body(poff_ref, xs_ref, ws_ref, wg_ref, wu_ref, wd_ref, ys_ref):
    e = pl.program_id(0)
    start = poff_ref[e]
    size = poff_ref[e + 1] - start
    nch = (size + CHUNK - 1) // CHUNK
    wg = wg_ref[0]
    wu = wu_ref[0]
    wd = wd_ref[0]

    def chunk(i, carry):
        base = pl.multiple_of(start + i * CHUNK, 8)
        rows = xs_ref[pl.ds(base, CHUNK), :]
        g = lax.dot_general(rows, wg, (((1,), (1,)), ((), ())),
                            preferred_element_type=jnp.float32)
        u = lax.dot_general(rows, wu, (((1,), (1,)), ((), ())),
                            preferred_element_type=jnp.float32)
        h = _gelu(g) * u
        part = lax.dot_general(h, wd, (((1,), (1,)), ((), ())),
                               preferred_element_type=jnp.float32)
        ys_ref[pl.ds(base, CHUNK), :] = part * ws_ref[pl.ds(base, CHUNK), :]
        return carry

    lax.fori_loop(0, nch, chunk, 0)


def _tc_grouped(poff, xs, ws, Wg, Wu, Wd):
    return pl.pallas_call(
        _tc_body,
        grid=(E,),
        in_specs=[
            pl.BlockSpec(memory_space=pltpu.SMEM),
            pl.BlockSpec((P, H), lambda e: (0, 0)),
            pl.BlockSpec((P, 1), lambda e: (0, 0)),
            pl.BlockSpec((1, I, H), lambda e: (e, 0, 0)),
            pl.BlockSpec((1, I, H), lambda e: (e, 0, 0)),
            pl.BlockSpec((1, H, I), lambda e: (e, 0, 0)),
        ],
        out_specs=pl.BlockSpec((P, H), lambda e: (0, 0)),
        out_shape=jax.ShapeDtypeStruct((P, H), jnp.float32),
        compiler_params=pltpu.CompilerParams(
            dimension_semantics=("arbitrary",)),
    )(poff, xs, ws, Wg, Wu, Wd)


def kernel(x, selected_experts, routing_weights, Wg, Wu, Wd):
    fe = selected_experts.reshape(-1).astype(jnp.int32)
    fw = routing_weights.reshape(-1).astype(jnp.float32)

    oh = (fe[:, None] == jnp.arange(E, dtype=jnp.int32)[None, :]).astype(jnp.int32)
    csum = jnp.cumsum(oh, axis=0)
    counts = csum[-1]
    rank = jnp.sum(oh * csum, axis=1) - 1
    pcounts = ((counts + 7) // 8) * 8
    poff = jnp.concatenate([jnp.zeros((1,), jnp.int32),
                            jnp.cumsum(pcounts).astype(jnp.int32)])
    pos = jnp.take(poff, fe) + rank

    tokw = jnp.stack([jnp.arange(T, dtype=jnp.float32), fw], axis=1)
    meta = jnp.full((P, 2), jnp.float32(TRASH)).at[pos].set(tokw)
    srcS = meta[:, 0].astype(jnp.int32)
    ws = meta[:, 1:2]
    ws = jnp.where(srcS[:, None] == TRASH, 0.0, ws)
    gsrc = jnp.where(srcS == TRASH, 0, srcS)
    dest = jnp.where(srcS == TRASH, TRASH + jnp.arange(P, dtype=jnp.int32), srcS)

    sc_gather, sc_scatter = _sc_kernels()
    xs = sc_gather(x, gsrc)
    ys = _tc_grouped(poff, xs, ws, Wg, Wu, Wd)
    out_ext = sc_scatter(ys, dest)
    return out_ext[:T]
